# Initial kernel scaffold; baseline (speedup 1.0000x reference)
import jax
import jax.numpy as jnp
from jax.experimental import pallas as pl


def kernel(t, x, edge_index, W, b, W_time):
    raise NotImplementedError("write your pallas kernel here")


# speedup vs baseline: 29.6643x; 29.6643x over previous
import functools

import jax
import jax.numpy as jnp
from jax import lax
from jax.experimental import pallas as pl
from jax.experimental.pallas import tpu as pltpu, tpu_sc as plsc

N = 10000
NPAD = 10240
E = 320000
D = 128
NW = 32
CHUNK = 125
ROWS_W = E // NW // CHUNK
SLAB = 8
NSLAB = ROWS_W // SLAB
SLICE = NPAD // 16

_mesh = plsc.VectorSubcoreMesh(core_axis_name="c", subcore_axis_name="s")

f32 = jnp.float32


@functools.partial(
    pl.kernel,
    out_type=[jax.ShapeDtypeStruct((NPAD,), f32),
              jax.ShapeDtypeStruct((NPAD,), f32)],
    mesh=_mesh,
    scratch_types=[
        pltpu.VMEM((ROWS_W, CHUNK), jnp.int32),
        pltpu.VMEM((SLICE,), f32),
        pltpu.VMEM_SHARED((NPAD,), f32),
    ],
)
def _sc_degree(dst_hbm, deg0_hbm, deg1_hbm, dst_v, buf_v, shared):
    cid = lax.axis_index("c")
    sid = lax.axis_index("s")
    wid = sid * 2 + cid

    def _zero(i, carry):
        buf_v[pl.ds(i * 16, 16)] = jnp.zeros((16,), f32)
        return carry
    lax.fori_loop(0, SLICE // 16, _zero, 0)

    pltpu.sync_copy(buf_v, shared.at[pl.ds(sid * SLICE, SLICE)])
    plsc.subcore_barrier()

    def _ones(i, carry):
        buf_v[pl.ds(i * 16, 16)] = jnp.ones((16,), f32)
        return carry
    lax.fori_loop(0, SLICE // 16 // 4, _ones, 0)

    pltpu.sync_copy(dst_hbm.at[wid], dst_v)

    def _scat(j, carry):
        pltpu.sync_copy(buf_v.at[pl.ds(0, CHUNK)],
                        shared.at[dst_v.at[j]], add=True)
        return carry
    lax.fori_loop(0, ROWS_W, _scat, 0)

    plsc.subcore_barrier()

    @pl.when(cid == 0)
    def _():
        pltpu.sync_copy(shared.at[pl.ds(sid * SLICE, SLICE)],
                        deg0_hbm.at[pl.ds(sid * SLICE, SLICE)])

    @pl.when(cid == 1)
    def _():
        pltpu.sync_copy(shared.at[pl.ds(sid * SLICE, SLICE)],
                        deg1_hbm.at[pl.ds(sid * SLICE, SLICE)])


@functools.partial(
    pl.kernel,
    out_type=[jax.ShapeDtypeStruct((NPAD, D), f32),
              jax.ShapeDtypeStruct((NPAD, D), f32)],
    mesh=_mesh,
    scratch_types=[
        pltpu.VMEM((SLAB, CHUNK), jnp.int32),
        pltpu.VMEM((SLAB, CHUNK), jnp.int32),
        pltpu.VMEM((CHUNK, D), f32),
        pltpu.VMEM_SHARED((NPAD, D), f32),
        pltpu.SemaphoreType.DMA,
    ],
)
def _sc_edges(src_hbm, dst_hbm, y_hbm, a0_hbm, a1_hbm,
              src_v, dst_v, rows_v, shared, gsem):
    cid = lax.axis_index("c")
    sid = lax.axis_index("s")
    wid = sid * 2 + cid

    def _zero(i, carry):
        rows_v[i // 8, pl.ds((i % 8) * 16, 16)] = jnp.zeros((16,), f32)
        return carry
    lax.fori_loop(0, 80 * 8, _zero, 0)

    def _zcopy(i, carry):
        pltpu.sync_copy(rows_v.at[pl.ds(0, 80)],
                        shared.at[pl.ds(sid * SLICE + i * 80, 80)])
        return carry
    lax.fori_loop(0, SLICE // 80, _zcopy, 0)
    plsc.subcore_barrier()

    def _slab(s, carry):
        pltpu.sync_copy(src_hbm.at[wid, pl.ds(s * SLAB, SLAB)], src_v)
        pltpu.sync_copy(dst_hbm.at[wid, pl.ds(s * SLAB, SLAB)], dst_v)

        def _edge(j, carry2):
            pltpu.async_copy(y_hbm.at[src_v.at[j]], rows_v, gsem).wait()
            pltpu.sync_copy(rows_v, shared.at[dst_v.at[j]], add=True)
            return carry2
        lax.fori_loop(0, SLAB, _edge, 0)
        return carry
    lax.fori_loop(0, NSLAB, _slab, 0)

    plsc.subcore_barrier()

    @pl.when(cid == 0)
    def _():
        pltpu.sync_copy(shared.at[pl.ds(sid * SLICE, SLICE)],
                        a0_hbm.at[pl.ds(sid * SLICE, SLICE)])

    @pl.when(cid == 1)
    def _():
        pltpu.sync_copy(shared.at[pl.ds(sid * SLICE, SLICE)],
                        a1_hbm.at[pl.ds(sid * SLICE, SLICE)])


def _tc_transform(xpad, wt, d0, d1):
    def body(x_ref, wt_ref, d0_ref, d1_ref, y_ref):
        deg = d0_ref[...] + d1_ref[...] + 1.0
        dinv = lax.rsqrt(deg)
        xw = jnp.dot(x_ref[...], wt_ref[...],
                     preferred_element_type=f32)
        y_ref[...] = xw * dinv

    return pl.pallas_call(
        body,
        out_shape=jax.ShapeDtypeStruct((NPAD, D), f32),
    )(xpad, wt, d0, d1)


def _tc_finish(a0, a1, y, d0, d1, b2, t2, wt2):
    def body(a0_ref, a1_ref, y_ref, d0_ref, d1_ref, b_ref, t_ref, w_ref,
             o_ref):
        deg = d0_ref[...] + d1_ref[...] + 1.0
        dinv = lax.rsqrt(deg)
        s = (a0_ref[...] + a1_ref[...] + y_ref[...]) * dinv + b_ref[...]
        tf = jax.nn.sigmoid(t_ref[...] * w_ref[...])
        o_ref[...] = jnp.maximum(s, 0.0) * tf

    return pl.pallas_call(
        body,
        out_shape=jax.ShapeDtypeStruct((NPAD, D), f32),
    )(a0, a1, y, d0, d1, b2, t2, wt2)


def kernel(t, x, edge_index, W, b, W_time):
    xpad = jnp.concatenate(
        [x, jnp.zeros((NPAD - N, D), f32)], axis=0)
    src3d = edge_index[0].reshape(NW, ROWS_W, CHUNK)
    dst3d = edge_index[1].reshape(NW, ROWS_W, CHUNK)
    wt = W.T

    deg0, deg1 = _sc_degree(dst3d)
    d0 = deg0.reshape(NPAD, 1)
    d1 = deg1.reshape(NPAD, 1)

    y = _tc_transform(xpad, wt, d0, d1)

    a0, a1 = _sc_edges(src3d, dst3d, y)

    b2 = b.reshape(1, D)
    t2 = t.reshape(1, 1)
    wt2 = W_time.reshape(1, D)

    out = _tc_finish(a0, a1, y, d0, d1, b2, t2, wt2)
    return out[:N]
